# baseline (speedup 1.0000x reference)
import functools

import jax
import jax.numpy as jnp
from jax import lax
from jax.experimental import pallas as pl
from jax.experimental.pallas import tpu as pltpu
from jax.experimental.pallas import tpu_sc as plsc

_V = 10000
_B = 1048576
_NW = 32
_BPW = _B // _NW
_CHUNK = 1024
_NCH = _BPW // _CHUNK
_GRP = _CHUNK // 16


def _sc_gather(idx, table):
    mesh = plsc.VectorSubcoreMesh(core_axis_name="c", subcore_axis_name="s")

    @functools.partial(
        pl.kernel,
        mesh=mesh,
        compiler_params=pltpu.CompilerParams(needs_layout_passes=False),
        out_type=[
            jax.ShapeDtypeStruct((_B * 4,), jnp.float32),
            jax.ShapeDtypeStruct((_B * 4,), jnp.float32),
        ],
        scratch_types=[
            pltpu.VMEM((_V * 6,), jnp.float32),
            pltpu.VMEM((_BPW,), jnp.int32),
            pltpu.VMEM((_CHUNK * 4,), jnp.float32),
            pltpu.VMEM((_CHUNK * 4,), jnp.float32),
            pltpu.VMEM((_CHUNK * 4,), jnp.float32),
            pltpu.VMEM((_CHUNK * 4,), jnp.float32),
            pltpu.SemaphoreType.DMA,
            pltpu.SemaphoreType.DMA,
            pltpu.SemaphoreType.DMA,
            pltpu.SemaphoreType.DMA,
            pltpu.SemaphoreType.DMA,
            pltpu.SemaphoreType.DMA,
        ],
    )
    def k(idx_hbm, tbl_hbm, oint_hbm, odir_hbm,
          tbl_v, idx_v, oi_v0, oi_v1, od_v0, od_v1,
          is0, is1, ois0, ois1, ods0, ods1):
        wid = lax.axis_index("s") * 2 + lax.axis_index("c")
        base = wid * _BPW
        oi_b = (oi_v0, oi_v1)
        od_b = (od_v0, od_v1)
        oisem = (ois0, ois1)
        odsem = (ods0, ods1)

        h_oi = [None, None]
        h_od = [None, None]
        h_idx = pltpu.async_copy(
            idx_hbm.at[pl.ds(base, _BPW)], idx_v, is0)
        h_dir = pltpu.async_copy(
            tbl_hbm.at[pl.ds(0, 3 * _V)], tbl_v.at[pl.ds(0, 3 * _V)], ods0)
        h_int = pltpu.async_copy(
            tbl_hbm.at[pl.ds(3 * _V, 3 * _V)],
            tbl_v.at[pl.ds(3 * _V, 3 * _V)], ods1)

        iota = lax.iota(jnp.int32, 16)
        zeros = jnp.zeros((16,), jnp.float32)

        @plsc.parallel_loop(0, _CHUNK * 4 // 16, unroll=8)
        def zbody(t):
            oi_v0[pl.ds(t * 16, 16)] = zeros
            oi_v1[pl.ds(t * 16, 16)] = zeros
            od_v0[pl.ds(t * 16, 16)] = zeros
            od_v1[pl.ds(t * 16, 16)] = zeros

        h_dir.wait()

        @plsc.parallel_loop(0, _V // 16, unroll=4)
        def nbody(t):
            x = tbl_v[pl.ds(t * 16, 16)]
            y = tbl_v[pl.ds(_V + t * 16, 16)]
            z = tbl_v[pl.ds(2 * _V + t * 16, 16)]
            s = x * x + y * y + z * z
            i = plsc.bitcast(s, jnp.int32)
            r = plsc.bitcast(0x5F3759DF - (i >> 1), jnp.float32)
            hs = 0.5 * s
            r = r * (1.5 - hs * r * r)
            r = r * (1.5 - hs * r * r)
            r = r * (1.5 - hs * r * r)
            r = jnp.minimum(r, 1e12)
            tbl_v[pl.ds(t * 16, 16)] = x * r
            tbl_v[pl.ds(_V + t * 16, 16)] = y * r
            tbl_v[pl.ds(2 * _V + t * 16, 16)] = z * r

        h_int.wait()
        h_idx.wait()
        for ch in range(_NCH):
            slot = ch % 2
            cbase = base + ch * _CHUNK
            coff = ch * _CHUNK
            oib, odb = oi_b[slot], od_b[slot]
            if ch >= 2:
                h_oi[slot].wait()
                h_od[slot].wait()

            @plsc.parallel_loop(0, _GRP, unroll=4)
            def body(g):
                iv = idx_v[pl.ds(coff + g * 16, 16)]
                p0 = iota + ((g >> 3) * 512 + (g & 7) * 16)
                p1 = p0 + 128
                p2 = p0 + 256
                plsc.store_scatter(odb, [p0], plsc.load_gather(tbl_v, [iv]))
                plsc.store_scatter(odb, [p1],
                                   plsc.load_gather(tbl_v, [iv + _V]))
                plsc.store_scatter(odb, [p2],
                                   plsc.load_gather(tbl_v, [iv + 2 * _V]))
                plsc.store_scatter(oib, [p0],
                                   plsc.load_gather(tbl_v, [iv + 3 * _V]))
                plsc.store_scatter(oib, [p1],
                                   plsc.load_gather(tbl_v, [iv + 4 * _V]))
                plsc.store_scatter(oib, [p2],
                                   plsc.load_gather(tbl_v, [iv + 5 * _V]))

            h_oi[slot] = pltpu.async_copy(
                oib, oint_hbm.at[pl.ds(cbase * 4, _CHUNK * 4)], oisem[slot])
            h_od[slot] = pltpu.async_copy(
                odb, odir_hbm.at[pl.ds(cbase * 4, _CHUNK * 4)], odsem[slot])
        for s in range(2):
            h_oi[s].wait()
            h_od[s].wait()

    return k(idx, table)


def _unpack(o):
    return (o.reshape(_B // 128, 4, 128)
            .transpose(0, 2, 1).reshape(_B, 4)[:, :3])


def kernel(rays_light_indices, light_dir, intensity):
    idx = rays_light_indices.astype(jnp.int32)
    table = jnp.concatenate(
        [light_dir.T.reshape(-1), intensity.T.reshape(-1)])
    oi, od = _sc_gather(idx, table)
    return _unpack(oi), _unpack(od)


# speedup vs baseline: 1.0923x; 1.0923x over previous
import functools

import jax
import jax.numpy as jnp
from jax import lax
from jax.experimental import pallas as pl
from jax.experimental.pallas import tpu as pltpu
from jax.experimental.pallas import tpu_sc as plsc

_V = 10000
_B = 1048576
_NW = 32
_BPW = _B // _NW
_CHUNK = 2048
_NCH = _BPW // _CHUNK
_GRP = _CHUNK // 16


def _sc_gather(idx, table):
    mesh = plsc.VectorSubcoreMesh(core_axis_name="c", subcore_axis_name="s")

    @functools.partial(
        pl.kernel,
        mesh=mesh,
        compiler_params=pltpu.CompilerParams(needs_layout_passes=False),
        out_type=[
            jax.ShapeDtypeStruct((_B * 4,), jnp.float32),
            jax.ShapeDtypeStruct((_B * 4,), jnp.float32),
        ],
        scratch_types=[
            pltpu.VMEM((_V * 6,), jnp.float32),
            pltpu.VMEM((_BPW,), jnp.int32),
            pltpu.VMEM((_CHUNK * 4,), jnp.float32),
            pltpu.VMEM((_CHUNK * 4,), jnp.float32),
            pltpu.VMEM((_CHUNK * 4,), jnp.float32),
            pltpu.VMEM((_CHUNK * 4,), jnp.float32),
            pltpu.SemaphoreType.DMA,
            pltpu.SemaphoreType.DMA,
            pltpu.SemaphoreType.DMA,
            pltpu.SemaphoreType.DMA,
            pltpu.SemaphoreType.DMA,
            pltpu.SemaphoreType.DMA,
        ],
    )
    def k(idx_hbm, tbl_hbm, oint_hbm, odir_hbm,
          tbl_v, idx_v, oi_v0, oi_v1, od_v0, od_v1,
          is0, is1, ois0, ois1, ods0, ods1):
        wid = lax.axis_index("s") * 2 + lax.axis_index("c")
        base = wid * _BPW
        oi_b = (oi_v0, oi_v1)
        od_b = (od_v0, od_v1)
        oisem = (ois0, ois1)
        odsem = (ods0, ods1)

        h_oi = [None, None]
        h_od = [None, None]
        h_idx = pltpu.async_copy(
            idx_hbm.at[pl.ds(base, _BPW)], idx_v, is0)
        h_dir = pltpu.async_copy(
            tbl_hbm.at[pl.ds(0, 3 * _V)], tbl_v.at[pl.ds(0, 3 * _V)], ods0)
        h_int = pltpu.async_copy(
            tbl_hbm.at[pl.ds(3 * _V, 3 * _V)],
            tbl_v.at[pl.ds(3 * _V, 3 * _V)], ods1)

        iota = lax.iota(jnp.int32, 16)
        zeros = jnp.zeros((16,), jnp.float32)

        @plsc.parallel_loop(0, _CHUNK * 4 // 16, unroll=8)
        def zbody(t):
            oi_v0[pl.ds(t * 16, 16)] = zeros
            oi_v1[pl.ds(t * 16, 16)] = zeros
            od_v0[pl.ds(t * 16, 16)] = zeros
            od_v1[pl.ds(t * 16, 16)] = zeros

        h_dir.wait()

        @plsc.parallel_loop(0, _V // 16, unroll=4)
        def nbody(t):
            x = tbl_v[pl.ds(t * 16, 16)]
            y = tbl_v[pl.ds(_V + t * 16, 16)]
            z = tbl_v[pl.ds(2 * _V + t * 16, 16)]
            s = x * x + y * y + z * z
            i = plsc.bitcast(s, jnp.int32)
            r = plsc.bitcast(0x5F3759DF - (i >> 1), jnp.float32)
            hs = 0.5 * s
            r = r * (1.5 - hs * r * r)
            r = r * (1.5 - hs * r * r)
            r = r * (1.5 - hs * r * r)
            r = jnp.minimum(r, 1e12)
            tbl_v[pl.ds(t * 16, 16)] = x * r
            tbl_v[pl.ds(_V + t * 16, 16)] = y * r
            tbl_v[pl.ds(2 * _V + t * 16, 16)] = z * r

        h_int.wait()
        h_idx.wait()
        for ch in range(_NCH):
            slot = ch % 2
            cbase = base + ch * _CHUNK
            coff = ch * _CHUNK
            oib, odb = oi_b[slot], od_b[slot]
            if ch >= 2:
                h_oi[slot].wait()
                h_od[slot].wait()

            @plsc.parallel_loop(0, _GRP, unroll=4)
            def body(g):
                iv = idx_v[pl.ds(coff + g * 16, 16)]
                p0 = iota + ((g >> 3) * 512 + (g & 7) * 16)
                p1 = p0 + 128
                p2 = p0 + 256
                plsc.store_scatter(odb, [p0], plsc.load_gather(tbl_v, [iv]))
                plsc.store_scatter(odb, [p1],
                                   plsc.load_gather(tbl_v, [iv + _V]))
                plsc.store_scatter(odb, [p2],
                                   plsc.load_gather(tbl_v, [iv + 2 * _V]))
                plsc.store_scatter(oib, [p0],
                                   plsc.load_gather(tbl_v, [iv + 3 * _V]))
                plsc.store_scatter(oib, [p1],
                                   plsc.load_gather(tbl_v, [iv + 4 * _V]))
                plsc.store_scatter(oib, [p2],
                                   plsc.load_gather(tbl_v, [iv + 5 * _V]))

            h_oi[slot] = pltpu.async_copy(
                oib, oint_hbm.at[pl.ds(cbase * 4, _CHUNK * 4)], oisem[slot])
            h_od[slot] = pltpu.async_copy(
                odb, odir_hbm.at[pl.ds(cbase * 4, _CHUNK * 4)], odsem[slot])
        for s in range(2):
            h_oi[s].wait()
            h_od[s].wait()

    return k(idx, table)


def _unpack(o):
    return (o.reshape(_B // 128, 4, 128)
            .transpose(0, 2, 1).reshape(_B, 4)[:, :3])


def kernel(rays_light_indices, light_dir, intensity):
    idx = rays_light_indices.astype(jnp.int32)
    table = jnp.concatenate(
        [light_dir.T.reshape(-1), intensity.T.reshape(-1)])
    oi, od = _sc_gather(idx, table)
    return _unpack(oi), _unpack(od)
